# baseline (speedup 1.0000x reference)
import jax
import jax.numpy as jnp
from jax.experimental import pallas as pl
from jax.experimental.pallas import tpu as pltpu

B = 64
D = 1024
MAX_SEQ = 256
L_TXT = MAX_SEQ - 4

NB = 8


def _body(t_ref, txt_ref, ctxt_ref, img_ref, pe_ref, fin_ref, g_ref, b_ref,
          out_ref):
    txt = txt_ref[...]

    tval = t_ref[...].astype(jnp.float32)
    k = jax.lax.broadcasted_iota(jnp.int32, (NB, D), 1)
    idx = jnp.where(k < D // 2, k, k - D // 2).astype(jnp.float32)
    inv_freq = jnp.exp(idx * (-jnp.log(10000.0) / (D // 2)))
    ang = tval * inv_freq
    temb = jnp.where(k < D // 2, jnp.sin(ang), jnp.cos(ang))

    bot = jnp.stack(
        [ctxt_ref[...], temb, img_ref[...],
         jnp.broadcast_to(fin_ref[...], (NB, D))], axis=0)

    pe = pe_ref[...][:, None, :]
    x = jnp.concatenate([txt, bot], axis=0) + pe

    mean = jnp.mean(x, axis=2, keepdims=True)
    xc = x - mean
    var = jnp.mean(xc * xc, axis=2, keepdims=True)
    y = (xc * jax.lax.rsqrt(var + 1e-5) * g_ref[...][None]
         + b_ref[...][None])
    out_ref[...] = y


@jax.jit
def kernel(clip_img_emb, t, encoded_txt, clip_txt_emb, pos_emb, final_emb,
           ln_gamma, ln_beta):
    grid = (B // NB,)
    out = pl.pallas_call(
        _body,
        grid=grid,
        in_specs=[
            pl.BlockSpec((NB, 1), lambda b: (b, 0)),
            pl.BlockSpec((L_TXT, NB, D), lambda b: (0, b, 0)),
            pl.BlockSpec((NB, D), lambda b: (b, 0)),
            pl.BlockSpec((NB, D), lambda b: (b, 0)),
            pl.BlockSpec((MAX_SEQ, D), lambda b: (0, 0)),
            pl.BlockSpec((1, D), lambda b: (0, 0)),
            pl.BlockSpec((1, D), lambda b: (0, 0)),
            pl.BlockSpec((1, D), lambda b: (0, 0)),
        ],
        out_specs=pl.BlockSpec((MAX_SEQ, NB, D), lambda b: (0, b, 0)),
        out_shape=jax.ShapeDtypeStruct((MAX_SEQ, B, D), jnp.float32),
        compiler_params=pltpu.CompilerParams(
            dimension_semantics=("parallel",)),
    )(t[:, None], encoded_txt.transpose(1, 0, 2), clip_txt_emb,
      clip_img_emb, pos_emb, final_emb[None, :], ln_gamma[None, :],
      ln_beta[None, :])
    return out.transpose(1, 0, 2)


# speedup vs baseline: 2.0794x; 2.0794x over previous
import jax
import jax.numpy as jnp
from jax.experimental import pallas as pl
from jax.experimental.pallas import tpu as pltpu

B = 64
D = 1024
MAX_SEQ = 256
L_TXT = MAX_SEQ - 4

NB = 8


def _body(t_ref, txt_ref, ctxt_ref, img_ref, pe_ref, fin_ref, g_ref, b_ref,
          out_ref):
    txt = txt_ref[...]

    tval = t_ref[...].astype(jnp.float32)
    k = jax.lax.broadcasted_iota(jnp.int32, (NB, D), 1)
    idx = jnp.where(k < D // 2, k, k - D // 2).astype(jnp.float32)
    inv_freq = jnp.exp(idx * (-jnp.log(10000.0) / (D // 2)))
    ang = tval * inv_freq
    temb = jnp.where(k < D // 2, jnp.sin(ang), jnp.cos(ang))

    bot = jnp.stack(
        [ctxt_ref[...], temb, img_ref[...],
         jnp.broadcast_to(fin_ref[...], (NB, D))], axis=0)

    pe = pe_ref[...][:, None, :]
    x = jnp.concatenate([txt, bot], axis=0) + pe

    mean = jnp.mean(x, axis=2, keepdims=True)
    xc = x - mean
    var = jnp.mean(xc * xc, axis=2, keepdims=True)
    y = (xc * jax.lax.rsqrt(var + 1e-5) * g_ref[...][None]
         + b_ref[...][None])
    out_ref[...] = jnp.transpose(y, (1, 0, 2))


@jax.jit
def kernel(clip_img_emb, t, encoded_txt, clip_txt_emb, pos_emb, final_emb,
           ln_gamma, ln_beta):
    grid = (B // NB,)
    out = pl.pallas_call(
        _body,
        grid=grid,
        in_specs=[
            pl.BlockSpec((NB, 1), lambda b: (b, 0)),
            pl.BlockSpec((L_TXT, NB, D), lambda b: (0, b, 0)),
            pl.BlockSpec((NB, D), lambda b: (b, 0)),
            pl.BlockSpec((NB, D), lambda b: (b, 0)),
            pl.BlockSpec((MAX_SEQ, D), lambda b: (0, 0)),
            pl.BlockSpec((1, D), lambda b: (0, 0)),
            pl.BlockSpec((1, D), lambda b: (0, 0)),
            pl.BlockSpec((1, D), lambda b: (0, 0)),
        ],
        out_specs=pl.BlockSpec((NB, MAX_SEQ, D), lambda b: (b, 0, 0)),
        out_shape=jax.ShapeDtypeStruct((B, MAX_SEQ, D), jnp.float32),
        compiler_params=pltpu.CompilerParams(
            dimension_semantics=("parallel",)),
    )(t[:, None], encoded_txt.transpose(1, 0, 2), clip_txt_emb,
      clip_img_emb, pos_emb, final_emb[None, :], ln_gamma[None, :],
      ln_beta[None, :])
    return out
